# Initial kernel scaffold; baseline (speedup 1.0000x reference)
import jax
import jax.numpy as jnp
from jax.experimental import pallas as pl


def kernel(x):
    raise NotImplementedError("write your pallas kernel here")


# speedup vs baseline: 8.0330x; 8.0330x over previous
import functools

import jax
import jax.numpy as jnp
from jax import lax
from jax.experimental import pallas as pl
from jax.experimental.pallas import tpu as pltpu
from jax.experimental.pallas import tpu_sc as plsc

B = 8
N = 2048
V_PAD = 1024
L = 16
CHUNKS = N // L


def _body(x_hbm, out_hbm, xv, ov, hist):
    c = lax.axis_index("c")
    s = lax.axis_index("s")

    @pl.when(s < B // 2)
    def _():
        row = c * (B // 2) + s
        pltpu.sync_copy(x_hbm.at[row], xv)
        for i in range(V_PAD // L):
            hist[pl.ds(i * L, L)] = jnp.zeros((L,), jnp.float32)

        def chunk(i, _):
            v = xv[pl.ds(i * L, L)]
            prev = plsc.load_gather(hist, [v])
            rank, last = plsc.scan_count(v)
            cnt = prev + rank.astype(jnp.float32)
            ov[pl.ds(i * L, L)] = cnt
            plsc.store_scatter(hist, [v], cnt, mask=last)
            return 0

        lax.fori_loop(0, CHUNKS, chunk, 0)
        pltpu.sync_copy(ov, out_hbm.at[row])


@jax.jit
def _counts(x):
    run = pl.kernel(
        _body,
        out_type=jax.ShapeDtypeStruct((B, N), jnp.float32),
        mesh=plsc.VectorSubcoreMesh(core_axis_name="c", subcore_axis_name="s"),
        scratch_types=[
            pltpu.VMEM((N,), jnp.int32),
            pltpu.VMEM((N,), jnp.float32),
            pltpu.VMEM((V_PAD,), jnp.float32),
        ],
        compiler_params=pltpu.CompilerParams(needs_layout_passes=False),
    )
    return run(x.astype(jnp.int32))


def kernel(x):
    return _counts(x)[..., None]


# speedup vs baseline: 8.3184x; 1.0355x over previous
import functools

import jax
import jax.numpy as jnp
from jax import lax
from jax.experimental import pallas as pl
from jax.experimental.pallas import tpu as pltpu
from jax.experimental.pallas import tpu_sc as plsc

B = 8
N = 2048
SEGS = 4
SEG = N // SEGS
V_PAD = 1024
L = 16
CHUNKS = SEG // L


def _body(x_hbm, out_hbm, xv, ov, hist, acc, tmp, spm):
    c = lax.axis_index("c")
    s = lax.axis_index("s")
    lrow = s // SEGS
    seg = s % SEGS
    row = c * (B // 2) + lrow
    base = row * N + seg * SEG

    pltpu.sync_copy(x_hbm.at[pl.ds(base, SEG)], xv)
    for i in range(V_PAD // L):
        hist[pl.ds(i * L, L)] = jnp.zeros((L,), jnp.float32)

    for i in range(CHUNKS):
        v = xv[pl.ds(i * L, L)]
        prev = plsc.load_gather(hist, [v])
        rank, last = plsc.scan_count(v)
        cnt = prev + rank.astype(jnp.float32)
        ov[pl.ds(i * L, L)] = cnt
        plsc.store_scatter(hist, [v], cnt, mask=last)

    pltpu.sync_copy(hist, spm.at[s])
    plsc.subcore_barrier()

    @pl.when(seg > 0)
    def _():
        pltpu.sync_copy(spm.at[s - seg], acc)
        for k in range(1, SEGS - 1):
            @pl.when(seg > k)
            def _():
                pltpu.sync_copy(spm.at[s - seg + k], tmp)
                for i in range(V_PAD // L):
                    d = pl.ds(i * L, L)
                    acc[d] = acc[d] + tmp[d]
        for i in range(CHUNKS):
            d = pl.ds(i * L, L)
            ov[d] = ov[d] + plsc.load_gather(acc, [xv[d]])

    pltpu.sync_copy(ov, out_hbm.at[pl.ds(base, SEG)])


@jax.jit
def _counts(x):
    run = pl.kernel(
        _body,
        out_type=jax.ShapeDtypeStruct((B * N,), jnp.float32),
        mesh=plsc.VectorSubcoreMesh(core_axis_name="c", subcore_axis_name="s"),
        scratch_types=[
            pltpu.VMEM((SEG,), jnp.int32),
            pltpu.VMEM((SEG,), jnp.float32),
            pltpu.VMEM((V_PAD,), jnp.float32),
            pltpu.VMEM((V_PAD,), jnp.float32),
            pltpu.VMEM((V_PAD,), jnp.float32),
            pltpu.VMEM_SHARED((16, V_PAD), jnp.float32),
        ],
        compiler_params=pltpu.CompilerParams(needs_layout_passes=False),
    )
    return run(x.astype(jnp.int32).reshape(B * N))


def kernel(x):
    return _counts(x).reshape(B, N, 1)
